# baseline (speedup 1.0000x reference)
import functools

import jax
import jax.numpy as jnp
from jax import lax
from jax.experimental import pallas as pl
from jax.experimental.pallas import tpu as pltpu
from jax.experimental.pallas import tpu_sc as plsc

_LAMBDA = 0.15
_B = 16384
_D = 64
_L = 16
_NC = 2
_NS = 16
_NW = _NC * _NS
_BPW = _B // _NW
_CH = 256
_NCH = _BPW // _CH
_GROUPS = _CH // _L

_mesh = plsc.VectorSubcoreMesh(core_axis_name="c", subcore_axis_name="s")


@functools.partial(
    pl.kernel,
    mesh=_mesh,
    compiler_params=pltpu.CompilerParams(needs_layout_passes=False),
    out_type=jax.ShapeDtypeStruct((_NW * _L,), jnp.float32),
    scratch_types=[
        pltpu.VMEM((_BPW,), jnp.int32),
        pltpu.VMEM((_CH, _D), jnp.float32),
        pltpu.VMEM((_D, _CH), jnp.float32),
        pltpu.VMEM((_L,), jnp.float32),
        pltpu.SemaphoreType.DMA,
        pltpu.SemaphoreType.DMA,
        pltpu.SemaphoreType.DMA,
    ],
)
def _sc_loss(feat_hbm, table_hbm, lab_hbm, out_hbm,
             lab_v, rows_v, feat_v, acc_v, sem_l, sem_f, sem_g):
    cid = lax.axis_index("c")
    sid = lax.axis_index("s")
    wid = sid * _NC + cid
    base = wid * _BPW

    pltpu.async_copy(lab_hbm.at[pl.ds(base, _BPW)], lab_v, sem_l).wait()

    lane = lax.iota(jnp.int32, _L)

    def make_group_body(ch):
        def group_body(g, acc):
            items = g * _L + lane
            s0 = jnp.zeros((_L,), jnp.float32)
            s1 = jnp.zeros((_L,), jnp.float32)
            s2 = jnp.zeros((_L,), jnp.float32)
            s3 = jnp.zeros((_L,), jnp.float32)
            parts = [s0, s1, s2, s3]
            for d in range(_D):
                dv = jnp.full((_L,), d, jnp.int32)
                f = feat_v[d, pl.ds(g * _L, _L)]
                p = plsc.load_gather(rows_v, [items, dv])
                df = f - p
                parts[d % 4] = parts[d % 4] + df * df
            x = (parts[0] + parts[1]) + (parts[2] + parts[3])
            i = lax.bitcast_convert_type(x, jnp.int32)
            i = jnp.int32(0x5F3759DF) - (i >> 1)
            y = lax.bitcast_convert_type(i, jnp.float32)
            for _ in range(3):
                y = y * (jnp.float32(1.5) - jnp.float32(0.5) * x * y * y)
            return acc + x * y
        return group_body

    acc = jnp.zeros((_L,), jnp.float32)
    for ch in range(_NCH):
        cbase = base + ch * _CH
        feat_cp = pltpu.async_copy(
            feat_hbm.at[:, pl.ds(cbase, _CH)], feat_v, sem_f)

        def issue(i, carry, _ch=ch):
            lvec = lab_v[pl.ds(_ch * _CH + i * _L, _L)]
            for u in range(_L):
                pltpu.async_copy(table_hbm.at[pl.ds(lvec[u], 1)],
                                 rows_v.at[pl.ds(i * _L + u, 1)], sem_g)
            return carry

        lax.fori_loop(0, _CH // _L, issue, jnp.int32(0))
        pltpu.make_async_copy(
            table_hbm.at[pl.ds(0, _CH)], rows_v, sem_g).wait()
        feat_cp.wait()
        acc = lax.fori_loop(0, _GROUPS, make_group_body(ch), acc)

    acc_v[...] = acc
    pltpu.sync_copy(acc_v, out_hbm.at[pl.ds(wid * _L, _L)])


def kernel(feature_prototypes, prototypes, labels):
    partials = _sc_loss(feature_prototypes.T, prototypes,
                        labels.astype(jnp.int32))
    return (_LAMBDA / _B) * jnp.sum(partials)


# speedup vs baseline: 1.1069x; 1.1069x over previous
import functools

import jax
import jax.numpy as jnp
from jax import lax
from jax.experimental import pallas as pl
from jax.experimental.pallas import tpu as pltpu
from jax.experimental.pallas import tpu_sc as plsc

_LAMBDA = 0.15
_B = 16384
_D = 64
_L = 16
_NC = 2
_NS = 16
_NW = _NC * _NS
_BPW = _B // _NW
_CH = 256
_NCH = _BPW // _CH
_GROUPS = _CH // _L

_mesh = plsc.VectorSubcoreMesh(core_axis_name="c", subcore_axis_name="s")


@functools.partial(
    pl.kernel,
    mesh=_mesh,
    out_type=jax.ShapeDtypeStruct((_NW * _L,), jnp.float32),
    scratch_types=[
        pltpu.VMEM((_BPW,), jnp.int32),
        pltpu.VMEM((_CH, _D), jnp.float32),
        pltpu.VMEM((_CH, _D), jnp.float32),
        pltpu.VMEM((_L,), jnp.float32),
        pltpu.SemaphoreType.DMA,
        pltpu.SemaphoreType.DMA,
        pltpu.SemaphoreType.DMA,
    ],
)
def _sc_loss(feat_hbm, table_hbm, lab_hbm, out_hbm,
             lab_v, rows_v, feat_v, acc_v, sem_l, sem_f, sem_g):
    cid = lax.axis_index("c")
    sid = lax.axis_index("s")
    wid = sid * _NC + cid
    base = wid * _BPW

    pltpu.async_copy(lab_hbm.at[pl.ds(base, _BPW)], lab_v, sem_l).wait()

    lane = lax.iota(jnp.int32, _L)
    lane_masks = [lane == jnp.int32(rr) for rr in range(_L)]
    shuffles = [jnp.bitwise_xor(lane, jnp.int32(k)) for k in (8, 4, 2, 1)]
    _dnums = lax.GatherDimensionNumbers(
        offset_dims=(), collapsed_slice_dims=(0,), start_index_map=(0,))

    def hsum_splat(v):
        for perm in shuffles:
            v = v + lax.gather(
                v, perm[:, None], dimension_numbers=_dnums,
                slice_sizes=(1,),
                mode=lax.GatherScatterMode.PROMISE_IN_BOUNDS)
        return v

    def make_group_body(ch):
        def group_body(g, acc):
            row0 = g * _L
            tot = jnp.zeros((_L,), jnp.float32)
            for rr in range(_L):
                r = row0 + rr
                parts = []
                for c in range(_D // _L):
                    f = feat_v[r, pl.ds(c * _L, _L)]
                    p = rows_v[r, pl.ds(c * _L, _L)]
                    df = f - p
                    parts.append(df * df)
                sq = (parts[0] + parts[1]) + (parts[2] + parts[3])
                tot = jnp.where(lane_masks[rr], hsum_splat(sq), tot)
            x = tot
            i = lax.bitcast_convert_type(x, jnp.int32)
            i = jnp.int32(0x5F3759DF) - (i >> 1)
            y = lax.bitcast_convert_type(i, jnp.float32)
            for _ in range(3):
                y = y * (jnp.float32(1.5) - jnp.float32(0.5) * x * y * y)
            return acc + x * y
        return group_body

    acc = jnp.zeros((_L,), jnp.float32)
    for ch in range(_NCH):
        cbase = base + ch * _CH
        feat_cp = pltpu.async_copy(
            feat_hbm.at[pl.ds(cbase, _CH)], feat_v, sem_f)

        def issue(i, carry, _ch=ch):
            lvec = lab_v[pl.ds(_ch * _CH + i * _L, _L)]
            for u in range(_L):
                pltpu.async_copy(table_hbm.at[pl.ds(lvec[u], 1)],
                                 rows_v.at[pl.ds(i * _L + u, 1)], sem_g)
            return carry

        lax.fori_loop(0, _CH // _L, issue, jnp.int32(0))
        pltpu.make_async_copy(
            table_hbm.at[pl.ds(0, _CH)], rows_v, sem_g).wait()
        feat_cp.wait()
        acc = lax.fori_loop(0, _GROUPS, make_group_body(ch), acc)

    acc_v[...] = acc
    pltpu.sync_copy(acc_v, out_hbm.at[pl.ds(wid * _L, _L)])


def kernel(feature_prototypes, prototypes, labels):
    partials = _sc_loss(feature_prototypes, prototypes,
                        labels.astype(jnp.int32))
    return (_LAMBDA / _B) * jnp.sum(partials)
